# Initial kernel scaffold; baseline (speedup 1.0000x reference)
import jax
import jax.numpy as jnp
from jax.experimental import pallas as pl


def kernel(x, pos_embed, gamma, beta):
    raise NotImplementedError("write your pallas kernel here")


# speedup vs baseline: 1.3179x; 1.3179x over previous
import jax
import jax.numpy as jnp
from jax.experimental import pallas as pl
from jax.experimental.pallas import tpu as pltpu

EPS = 1e-5
BT_BLK = 8


def _ln_kernel(x_ref, pos_ref, gamma_ref, beta_ref, out_ref):
    h = x_ref[...] + pos_ref[...][None, :, :]
    mean = jnp.mean(h, axis=-1, keepdims=True)
    d = h - mean
    var = jnp.mean(d * d, axis=-1, keepdims=True)
    h_norm = d * jax.lax.rsqrt(var + EPS)
    out_ref[...] = h_norm * gamma_ref[...][None, None, :] + beta_ref[...][None, None, :]


def kernel(x, pos_embed, gamma, beta):
    B, T, N, D = x.shape
    xf = x.reshape(B * T, N, D)
    grid = (B * T) // BT_BLK
    out = pl.pallas_call(
        _ln_kernel,
        grid=(grid,),
        in_specs=[
            pl.BlockSpec((BT_BLK, N, D), lambda i: (i, 0, 0)),
            pl.BlockSpec((N, D), lambda i: (0, 0)),
            pl.BlockSpec((D,), lambda i: (0,)),
            pl.BlockSpec((D,), lambda i: (0,)),
        ],
        out_specs=pl.BlockSpec((BT_BLK, N, D), lambda i: (i, 0, 0)),
        out_shape=jax.ShapeDtypeStruct((B * T, N, D), x.dtype),
        compiler_params=pltpu.CompilerParams(
            dimension_semantics=("arbitrary",),
        ),
    )(xf, pos_embed, gamma, beta)
    return out.reshape(B, T, N, D)


# speedup vs baseline: 1.6901x; 1.2824x over previous
import jax
import jax.numpy as jnp
from jax.experimental import pallas as pl
from jax.experimental.pallas import tpu as pltpu

EPS = 1e-5
BT_BLK = 16


def _ln_kernel(x_ref, pos_ref, gamma_ref, beta_ref, out_ref):
    h = x_ref[...] + pos_ref[...][None, :, :]
    mean = jnp.mean(h, axis=-1, keepdims=True)
    d = h - mean
    var = jnp.mean(d * d, axis=-1, keepdims=True)
    h_norm = d * jax.lax.rsqrt(var + EPS)
    out_ref[...] = h_norm * gamma_ref[...][None, None, :] + beta_ref[...][None, None, :]


def kernel(x, pos_embed, gamma, beta):
    B, T, N, D = x.shape
    xf = x.reshape(B * T, N, D)
    grid = (B * T) // BT_BLK
    out = pl.pallas_call(
        _ln_kernel,
        grid=(grid,),
        in_specs=[
            pl.BlockSpec((BT_BLK, N, D), lambda i: (i, 0, 0)),
            pl.BlockSpec((N, D), lambda i: (0, 0)),
            pl.BlockSpec((D,), lambda i: (0,)),
            pl.BlockSpec((D,), lambda i: (0,)),
        ],
        out_specs=pl.BlockSpec((BT_BLK, N, D), lambda i: (i, 0, 0)),
        out_shape=jax.ShapeDtypeStruct((B * T, N, D), x.dtype),
        compiler_params=pltpu.CompilerParams(
            dimension_semantics=("arbitrary",),
        ),
    )(xf, pos_embed, gamma, beta)
    return out.reshape(B, T, N, D)


# speedup vs baseline: 2.0064x; 1.1871x over previous
import jax
import jax.numpy as jnp
from jax.experimental import pallas as pl
from jax.experimental.pallas import tpu as pltpu

EPS = 1e-5
BT_BLK = 32


def _ln_kernel(x_ref, pos_ref, gamma_ref, beta_ref, out_ref):
    h = x_ref[...] + pos_ref[...][None, :, :]
    mean = jnp.mean(h, axis=-1, keepdims=True)
    d = h - mean
    var = jnp.mean(d * d, axis=-1, keepdims=True)
    h_norm = d * jax.lax.rsqrt(var + EPS)
    out_ref[...] = h_norm * gamma_ref[...][None, None, :] + beta_ref[...][None, None, :]


def kernel(x, pos_embed, gamma, beta):
    B, T, N, D = x.shape
    xf = x.reshape(B * T, N, D)
    grid = (B * T) // BT_BLK
    out = pl.pallas_call(
        _ln_kernel,
        grid=(grid,),
        in_specs=[
            pl.BlockSpec((BT_BLK, N, D), lambda i: (i, 0, 0)),
            pl.BlockSpec((N, D), lambda i: (0, 0)),
            pl.BlockSpec((D,), lambda i: (0,)),
            pl.BlockSpec((D,), lambda i: (0,)),
        ],
        out_specs=pl.BlockSpec((BT_BLK, N, D), lambda i: (i, 0, 0)),
        out_shape=jax.ShapeDtypeStruct((B * T, N, D), x.dtype),
        compiler_params=pltpu.CompilerParams(
            dimension_semantics=("arbitrary",),
        ),
    )(xf, pos_embed, gamma, beta)
    return out.reshape(B, T, N, D)


# speedup vs baseline: 2.1508x; 1.0720x over previous
import jax
import jax.numpy as jnp
from jax.experimental import pallas as pl
from jax.experimental.pallas import tpu as pltpu

EPS = 1e-5
BT_BLK = 64


def _ln_kernel(x_ref, pos_ref, gamma_ref, beta_ref, out_ref):
    h = x_ref[...] + pos_ref[...][None, :, :]
    mean = jnp.mean(h, axis=-1, keepdims=True)
    d = h - mean
    var = jnp.mean(d * d, axis=-1, keepdims=True)
    h_norm = d * jax.lax.rsqrt(var + EPS)
    out_ref[...] = h_norm * gamma_ref[...][None, None, :] + beta_ref[...][None, None, :]


def kernel(x, pos_embed, gamma, beta):
    B, T, N, D = x.shape
    xf = x.reshape(B * T, N, D)
    grid = (B * T) // BT_BLK
    out = pl.pallas_call(
        _ln_kernel,
        grid=(grid,),
        in_specs=[
            pl.BlockSpec((BT_BLK, N, D), lambda i: (i, 0, 0)),
            pl.BlockSpec((N, D), lambda i: (0, 0)),
            pl.BlockSpec((D,), lambda i: (0,)),
            pl.BlockSpec((D,), lambda i: (0,)),
        ],
        out_specs=pl.BlockSpec((BT_BLK, N, D), lambda i: (i, 0, 0)),
        out_shape=jax.ShapeDtypeStruct((B * T, N, D), x.dtype),
        compiler_params=pltpu.CompilerParams(
            dimension_semantics=("arbitrary",),
        ),
    )(xf, pos_embed, gamma, beta)
    return out.reshape(B, T, N, D)
